# Initial kernel scaffold; baseline (speedup 1.0000x reference)
import jax
import jax.numpy as jnp
from jax.experimental import pallas as pl


def kernel(feat, W_theta, b_theta, W_phi, b_phi):
    raise NotImplementedError("write your pallas kernel here")


# speedup vs baseline: 3.3134x; 3.3134x over previous
import functools

import jax
import jax.numpy as jnp
from jax import lax
from jax.experimental import pallas as pl
from jax.experimental.pallas import tpu as pltpu
from jax.experimental.pallas import tpu_sc as plsc

N = 10000
D = 128
K = 16
NPAD = 10240
RQ = 256

NW = 32
NPW = NPAD // NW
BN = 8
NIT = NPW // BN


def _knn_body(featq_ref, featT_ref, wtT_ref, wpT_ref, bsum_ref,
              idx_ref, t0_ref, base_ref, dist_ref):
    q = featq_ref[...]
    kT = featT_ref[...]
    t0 = jnp.dot(q, wtT_ref[...], preferred_element_type=jnp.float32)
    ph = jnp.dot(q, wpT_ref[...], preferred_element_type=jnp.float32)
    t0_ref[...] = t0
    base_ref[...] = t0 + ph + bsum_ref[...]

    sqk = jnp.sum(kT * kT, axis=0, keepdims=True)
    colv = lax.broadcasted_iota(jnp.int32, (1, NPAD), 1)
    d = sqk - 2.0 * jnp.dot(q, kT, preferred_element_type=jnp.float32)
    dist_ref[...] = jnp.where(colv >= N, jnp.float32(1e30), d)

    cols = lax.broadcasted_iota(jnp.int32, (RQ, NPAD), 1)
    m = jnp.full((RQ, 1), -jnp.inf, jnp.float32)
    pi = jnp.full((RQ, 1), -1, jnp.int32)
    picks = []
    for _ in range(K):
        dd = dist_ref[...]
        ok = (dd > m) | ((dd == m) & (cols > pi))
        mt = jnp.min(jnp.where(ok, dd, jnp.float32(jnp.inf)),
                     axis=1, keepdims=True)
        it = jnp.min(jnp.where(ok & (dd == mt), cols, jnp.int32(2 ** 30)),
                     axis=1, keepdims=True)
        picks.append(it)
        m, pi = mt, it
    idx_ref[...] = jnp.concatenate(picks, axis=1)


def _knn_call(featp, featT, wtT, wpT, bsum):
    return pl.pallas_call(
        _knn_body,
        grid=(NPAD // RQ,),
        in_specs=[
            pl.BlockSpec((RQ, D), lambda i: (i, 0)),
            pl.BlockSpec((D, NPAD), lambda i: (0, 0)),
            pl.BlockSpec((D, D), lambda i: (0, 0)),
            pl.BlockSpec((D, D), lambda i: (0, 0)),
            pl.BlockSpec((1, D), lambda i: (0, 0)),
        ],
        out_specs=[
            pl.BlockSpec((RQ, K), lambda i: (i, 0)),
            pl.BlockSpec((RQ, D), lambda i: (i, 0)),
            pl.BlockSpec((RQ, D), lambda i: (i, 0)),
        ],
        out_shape=[
            jax.ShapeDtypeStruct((NPAD, K), jnp.int32),
            jax.ShapeDtypeStruct((NPAD, D), jnp.float32),
            jax.ShapeDtypeStruct((NPAD, D), jnp.float32),
        ],
        scratch_shapes=[pltpu.VMEM((RQ, NPAD), jnp.float32)],
    )(featp, featT, wtT, wpT, bsum)


def _sc_body(t0_hbm, base_hbm, idx_hbm, out_hbm, idxv, rows, basev, outv, sem):
    nc = plsc.get_sparse_core_info().num_cores
    wid = lax.axis_index("s") * nc + lax.axis_index("c")

    def body(g, carry):
        nb = wid * NPW + g * BN
        pltpu.sync_copy(idx_hbm.at[pl.ds(nb * K, BN * K)], idxv)
        pltpu.async_copy(t0_hbm.at[idxv], rows, sem).wait()
        pltpu.sync_copy(base_hbm.at[pl.ds(nb, BN)], basev)
        for b in range(BN):
            for c in range(D // 16):
                sl = pl.ds(c * 16, 16)
                acc = rows[b * K, sl]
                for j in range(1, K):
                    acc = jnp.minimum(acc, rows[b * K + j, sl])
                outv[b, sl] = basev[b, sl] - acc
        pltpu.sync_copy(outv, out_hbm.at[pl.ds(nb, BN)])
        return carry

    lax.fori_loop(0, NIT, body, 0)


@functools.cache
def _sc_gather_min():
    return pl.kernel(
        _sc_body,
        out_type=jax.ShapeDtypeStruct((NPAD, D), jnp.float32),
        mesh=plsc.VectorSubcoreMesh(core_axis_name="c", subcore_axis_name="s"),
        scratch_types=[
            pltpu.VMEM((BN * K,), jnp.int32),
            pltpu.VMEM((BN * K, D), jnp.float32),
            pltpu.VMEM((BN, D), jnp.float32),
            pltpu.VMEM((BN, D), jnp.float32),
            pltpu.SemaphoreType.DMA,
        ],
    )


@jax.jit
def kernel(feat, W_theta, b_theta, W_phi, b_phi):
    featp = jnp.pad(feat, ((0, NPAD - N), (0, 0)))
    featT = featp.T
    bsum = (b_theta + b_phi).reshape(1, D)
    idx, t0, base = _knn_call(featp, featT, W_theta.T, W_phi.T, bsum)
    out = _sc_gather_min()(t0, base, idx.reshape(-1))
    return out[:N]
